# baseline (speedup 1.0000x reference)
import functools

import jax
import jax.numpy as jnp
import numpy as np
from jax import lax
from jax.experimental import pallas as pl
from jax.experimental.pallas import tpu as pltpu
from jax.experimental.pallas import tpu_sc as plsc

_NC = 2
_NS = 16
_NW = _NC * _NS

_N_ROWS = 8192
_DIM = 128
_ROWS_PER_W = _N_ROWS // _NW
_CHUNK = 32
_N_CHUNK = _ROWS_PER_W // _CHUNK

@functools.cache
def _sc_gather_kernel(n_rows):
    rows_per_w = n_rows // _NW
    n_chunk = rows_per_w // _CHUNK
    mesh = plsc.VectorSubcoreMesh(core_axis_name="c", subcore_axis_name="s")

    @functools.partial(
        pl.kernel,
        out_type=jax.ShapeDtypeStruct((n_rows, _DIM), jnp.float32),
        mesh=mesh,
        scratch_types=[
            pltpu.VMEM((n_chunk, _CHUNK), jnp.int32),
            pltpu.VMEM((rows_per_w, _DIM), jnp.float32),
            pltpu.SemaphoreType.DMA,
        ],
    )
    def _sc_gather(table_hbm, idx_hbm, out_hbm, idx_v, rows_v, sem):
        wid = lax.axis_index("s") * _NC + lax.axis_index("c")
        pltpu.sync_copy(idx_hbm.at[pl.ds(wid * n_chunk, n_chunk)], idx_v)
        cps = [
            pltpu.async_copy(
                table_hbm.at[idx_v.at[j]],
                rows_v.at[pl.ds(j * _CHUNK, _CHUNK)],
                sem,
            )
            for j in range(n_chunk)
        ]
        for cp in cps:
            cp.wait()
        pltpu.sync_copy(rows_v,
                        out_hbm.at[pl.ds(wid * rows_per_w, rows_per_w)])

    return _sc_gather


_BLK = 512
_SEQ_BLK = _BLK // 4

def _tc_body(emb_ref, w_ref, pe_ref, b_ref, sc_ref, o_ref, sel_ref, red_ref):
    @pl.when(pl.program_id(0) == 0)
    def _():
        r_io = lax.broadcasted_iota(jnp.int32, (_BLK, _SEQ_BLK), 0)
        q_io = lax.broadcasted_iota(jnp.int32, (_BLK, _SEQ_BLK), 1)
        sel_ref[...] = (r_io // 4 == q_io).astype(jnp.float32)
        k_io = lax.broadcasted_iota(jnp.int32, (_DIM, _DIM), 0)
        m_io = lax.broadcasted_iota(jnp.int32, (_DIM, _DIM), 1)
        first = k_io == 0
        red_m = jnp.where(m_io == 0, jnp.where(first, 1.0, 0.0), 0.0)
        red_ref[...] = jnp.where(m_io == 1, jnp.where(first, 0.0, 1.0),
                                 red_m)

    x = lax.dot_general(emb_ref[...], w_ref[...], (((1,), (1,)), ((), ())),
                        preferred_element_type=jnp.float32)
    x = x + jnp.dot(sel_ref[...], pe_ref[...],
                    preferred_element_type=jnp.float32)
    x = x + b_ref[...]
    c_io = lax.broadcasted_iota(jnp.int32, (_BLK, _DIM), 1)
    red = jnp.dot(jnp.where(c_io == 0, x, x * x), red_ref[...],
                  preferred_element_type=jnp.float32)
    x0 = red[:, :1]
    denom = jnp.maximum(red[:, 1:2], 1e-8)
    esc = jnp.exp(sc_ref[0, 0])
    t = esc / (1.0 + jnp.exp(-x0)) + 1.1
    s = (t * t - 1.0) / denom
    o_ref[...] = jnp.where(c_io == 0, t, x * jnp.sqrt(s))


def _tc_compute(emb, w, pe, b2, scl):
    return pl.pallas_call(
        _tc_body,
        grid=(_N_ROWS // _BLK,),
        in_specs=[
            pl.BlockSpec((_BLK, _DIM), lambda i: (i, 0)),
            pl.BlockSpec((_DIM, _DIM), lambda i: (0, 0)),
            pl.BlockSpec((_SEQ_BLK, _DIM), lambda i: (i, 0)),
            pl.BlockSpec((1, _DIM), lambda i: (0, 0)),
            pl.BlockSpec(memory_space=pltpu.SMEM),
        ],
        out_specs=pl.BlockSpec((_BLK, _DIM), lambda i: (i, 0)),
        out_shape=jax.ShapeDtypeStruct((_N_ROWS, _DIM), jnp.float32),
        scratch_shapes=[
            pltpu.VMEM((_BLK, _SEQ_BLK), jnp.float32),
            pltpu.VMEM((_DIM, _DIM), jnp.float32),
        ],
    )(emb, w, pe, b2, scl)


def kernel(source, embedding, pe, W, b, scale):
    seq, batch, nfeat = source.shape
    n = seq * batch * nfeat
    idx = source.reshape(n // _CHUNK, _CHUNK).astype(jnp.int32)
    rows = _sc_gather_kernel(n)(embedding, idx)
    out = _tc_compute(rows, W, pe[:seq].reshape(seq, _DIM),
                      b.reshape(1, _DIM), scale.reshape(1, 1))
    return out.reshape(seq, batch, _DIM)


# speedup vs baseline: 1.1338x; 1.1338x over previous
import functools

import jax
import jax.numpy as jnp
import numpy as np
from jax import lax
from jax.experimental import pallas as pl
from jax.experimental.pallas import tpu as pltpu
from jax.experimental.pallas import tpu_sc as plsc

_NC = 2
_NS = 16
_NW = _NC * _NS

_N_ROWS = 8192
_DIM = 128
_ROWS_PER_W = _N_ROWS // _NW
_CHUNK = 32
_N_CHUNK = _ROWS_PER_W // _CHUNK

@functools.cache
def _sc_gather_kernel(n_rows):
    rows_per_w = n_rows // _NW
    n_chunk = rows_per_w // _CHUNK
    mesh = plsc.VectorSubcoreMesh(core_axis_name="c", subcore_axis_name="s")

    @functools.partial(
        pl.kernel,
        out_type=jax.ShapeDtypeStruct((n_rows, _DIM), jnp.float32),
        mesh=mesh,
        scratch_types=[
            pltpu.VMEM((n_chunk, _CHUNK), jnp.int32),
            pltpu.VMEM((rows_per_w, _DIM), jnp.float32),
            pltpu.SemaphoreType.DMA,
        ],
    )
    def _sc_gather(table_hbm, idx_hbm, out_hbm, idx_v, rows_v, sem):
        wid = lax.axis_index("s") * _NC + lax.axis_index("c")
        pltpu.sync_copy(idx_hbm.at[pl.ds(wid * n_chunk, n_chunk)], idx_v)
        cps = [
            pltpu.async_copy(
                table_hbm.at[idx_v.at[j]],
                rows_v.at[pl.ds(j * _CHUNK, _CHUNK)],
                sem,
            )
            for j in range(n_chunk)
        ]
        for cp in cps:
            cp.wait()
        pltpu.sync_copy(rows_v,
                        out_hbm.at[pl.ds(wid * rows_per_w, rows_per_w)])

    return _sc_gather


_BLK = 1024
_SEQ_BLK = _BLK // 4

def _tc_body(emb_ref, w_ref, pe_ref, b_ref, sc_ref, o_ref, sel_ref, red_ref):
    @pl.when(pl.program_id(0) == 0)
    def _():
        r_io = lax.broadcasted_iota(jnp.int32, (_BLK, _SEQ_BLK), 0)
        q_io = lax.broadcasted_iota(jnp.int32, (_BLK, _SEQ_BLK), 1)
        sel_ref[...] = (r_io // 4 == q_io).astype(jnp.float32)
        k_io = lax.broadcasted_iota(jnp.int32, (_DIM, _DIM), 0)
        m_io = lax.broadcasted_iota(jnp.int32, (_DIM, _DIM), 1)
        first = k_io == 0
        red_m = jnp.where(m_io == 0, jnp.where(first, 1.0, 0.0), 0.0)
        red_ref[...] = jnp.where(m_io == 1, jnp.where(first, 0.0, 1.0),
                                 red_m)

    x = lax.dot_general(emb_ref[...], w_ref[...], (((1,), (1,)), ((), ())),
                        preferred_element_type=jnp.float32)
    x = x + jnp.dot(sel_ref[...], pe_ref[...],
                    preferred_element_type=jnp.float32)
    x = x + b_ref[...]
    c_io = lax.broadcasted_iota(jnp.int32, (_BLK, _DIM), 1)
    red = jnp.dot(jnp.where(c_io == 0, x, x * x), red_ref[...],
                  preferred_element_type=jnp.float32)
    x0 = red[:, :1]
    denom = jnp.maximum(red[:, 1:2], 1e-8)
    esc = jnp.exp(sc_ref[0, 0])
    t = esc / (1.0 + jnp.exp(-x0)) + 1.1
    s = (t * t - 1.0) / denom
    o_ref[...] = jnp.where(c_io == 0, t, x * jnp.sqrt(s))


def _tc_compute(emb, w, pe, b2, scl):
    return pl.pallas_call(
        _tc_body,
        grid=(_N_ROWS // _BLK,),
        in_specs=[
            pl.BlockSpec((_BLK, _DIM), lambda i: (i, 0)),
            pl.BlockSpec((_DIM, _DIM), lambda i: (0, 0)),
            pl.BlockSpec((_SEQ_BLK, _DIM), lambda i: (i, 0)),
            pl.BlockSpec((1, _DIM), lambda i: (0, 0)),
            pl.BlockSpec(memory_space=pltpu.SMEM),
        ],
        out_specs=pl.BlockSpec((_BLK, _DIM), lambda i: (i, 0)),
        out_shape=jax.ShapeDtypeStruct((_N_ROWS, _DIM), jnp.float32),
        scratch_shapes=[
            pltpu.VMEM((_BLK, _SEQ_BLK), jnp.float32),
            pltpu.VMEM((_DIM, _DIM), jnp.float32),
        ],
    )(emb, w, pe, b2, scl)


def kernel(source, embedding, pe, W, b, scale):
    seq, batch, nfeat = source.shape
    n = seq * batch * nfeat
    idx = source.reshape(n // _CHUNK, _CHUNK).astype(jnp.int32)
    rows = _sc_gather_kernel(n)(embedding, idx)
    out = _tc_compute(rows, W, pe[:seq].reshape(seq, _DIM),
                      b.reshape(1, _DIM), scale.reshape(1, 1))
    return out.reshape(seq, batch, _DIM)


# speedup vs baseline: 1.1911x; 1.0506x over previous
import functools

import jax
import jax.numpy as jnp
import numpy as np
from jax import lax
from jax.experimental import pallas as pl
from jax.experimental.pallas import tpu as pltpu
from jax.experimental.pallas import tpu_sc as plsc

_NC = 2
_NS = 16
_NW = _NC * _NS

_N_ROWS = 8192
_DIM = 128
_ROWS_PER_W = _N_ROWS // _NW
_CHUNK = 128
_N_CHUNK = _ROWS_PER_W // _CHUNK

@functools.cache
def _sc_gather_kernel(n_rows):
    rows_per_w = n_rows // _NW
    n_chunk = rows_per_w // _CHUNK
    mesh = plsc.VectorSubcoreMesh(core_axis_name="c", subcore_axis_name="s")

    @functools.partial(
        pl.kernel,
        out_type=jax.ShapeDtypeStruct((n_rows, _DIM), jnp.float32),
        mesh=mesh,
        scratch_types=[
            pltpu.VMEM((n_chunk, _CHUNK), jnp.int32),
            pltpu.VMEM((rows_per_w, _DIM), jnp.float32),
            pltpu.SemaphoreType.DMA,
        ],
    )
    def _sc_gather(table_hbm, idx_hbm, out_hbm, idx_v, rows_v, sem):
        wid = lax.axis_index("s") * _NC + lax.axis_index("c")
        pltpu.sync_copy(idx_hbm.at[pl.ds(wid * n_chunk, n_chunk)], idx_v)
        cps = [
            pltpu.async_copy(
                table_hbm.at[idx_v.at[j]],
                rows_v.at[pl.ds(j * _CHUNK, _CHUNK)],
                sem,
            )
            for j in range(n_chunk)
        ]
        for cp in cps:
            cp.wait()
        pltpu.sync_copy(rows_v,
                        out_hbm.at[pl.ds(wid * rows_per_w, rows_per_w)])

    return _sc_gather


_BLK = 2048
_SEQ_BLK = _BLK // 4
_SEL_R = 1024
_SEL_C = _SEL_R // 4

def _tc_body(emb_ref, w_ref, pe_ref, b_ref, sc_ref, o_ref, sel_ref, red_ref):
    @pl.when(pl.program_id(0) == 0)
    def _():
        r_io = lax.broadcasted_iota(jnp.int32, (_SEL_R, _SEL_C), 0)
        q_io = lax.broadcasted_iota(jnp.int32, (_SEL_R, _SEL_C), 1)
        sel_ref[...] = (r_io // 4 == q_io).astype(jnp.float32)
        k_io = lax.broadcasted_iota(jnp.int32, (_DIM, _DIM), 0)
        m_io = lax.broadcasted_iota(jnp.int32, (_DIM, _DIM), 1)
        first = k_io == 0
        red_m = jnp.where(m_io == 0, jnp.where(first, 1.0, 0.0), 0.0)
        red_ref[...] = jnp.where(m_io == 1, jnp.where(first, 0.0, 1.0),
                                 red_m)

    x = lax.dot_general(emb_ref[...], w_ref[...], (((1,), (1,)), ((), ())),
                        preferred_element_type=jnp.float32)
    sel = sel_ref[...]
    pe_all = pe_ref[...]
    xpe = jnp.concatenate(
        [jnp.dot(sel, pe_all[h * _SEL_C:(h + 1) * _SEL_C],
                 preferred_element_type=jnp.float32)
         for h in range(_BLK // _SEL_R)],
        axis=0,
    )
    x = x + xpe
    x = x + b_ref[...]
    c_io = lax.broadcasted_iota(jnp.int32, (_BLK, _DIM), 1)
    red = jnp.dot(jnp.where(c_io == 0, x, x * x), red_ref[...],
                  preferred_element_type=jnp.float32)
    x0 = red[:, :1]
    denom = jnp.maximum(red[:, 1:2], 1e-8)
    esc = jnp.exp(sc_ref[0, 0])
    t = esc / (1.0 + jnp.exp(-x0)) + 1.1
    s = (t * t - 1.0) / denom
    o_ref[...] = jnp.where(c_io == 0, t, x * jnp.sqrt(s))


def _tc_compute(emb, w, pe, b2, scl):
    return pl.pallas_call(
        _tc_body,
        grid=(_N_ROWS // _BLK,),
        in_specs=[
            pl.BlockSpec((_BLK, _DIM), lambda i: (i, 0)),
            pl.BlockSpec((_DIM, _DIM), lambda i: (0, 0)),
            pl.BlockSpec((_SEQ_BLK, _DIM), lambda i: (i, 0)),
            pl.BlockSpec((1, _DIM), lambda i: (0, 0)),
            pl.BlockSpec(memory_space=pltpu.SMEM),
        ],
        out_specs=pl.BlockSpec((_BLK, _DIM), lambda i: (i, 0)),
        out_shape=jax.ShapeDtypeStruct((_N_ROWS, _DIM), jnp.float32),
        scratch_shapes=[
            pltpu.VMEM((_SEL_R, _SEL_C), jnp.float32),
            pltpu.VMEM((_DIM, _DIM), jnp.float32),
        ],
    )(emb, w, pe, b2, scl)


def kernel(source, embedding, pe, W, b, scale):
    seq, batch, nfeat = source.shape
    n = seq * batch * nfeat
    idx = source.reshape(n // _CHUNK, _CHUNK).astype(jnp.int32)
    rows = _sc_gather_kernel(n)(embedding, idx)
    out = _tc_compute(rows, W, pe[:seq].reshape(seq, _DIM),
                      b.reshape(1, _DIM), scale.reshape(1, 1))
    return out.reshape(seq, batch, _DIM)
